# Initial kernel scaffold; baseline (speedup 1.0000x reference)
import jax
import jax.numpy as jnp
from jax.experimental import pallas as pl


def kernel(x, edge_index, edge_attr, W_edge, b_edge, t, mlp_W1, mlp_b1, mlp_W2, mlp_b2, bn1_gamma, bn1_beta, lin_W, bn2_gamma, bn2_beta):
    raise NotImplementedError("write your pallas kernel here")


# speedup vs baseline: 2.2061x; 2.2061x over previous
import functools

import jax
import jax.numpy as jnp
from jax import lax
from jax.experimental import pallas as pl
from jax.experimental.pallas import tpu as pltpu
from jax.experimental.pallas import tpu_sc as plsc

N = 10000
E = 320000
D = 128
DH = 64
DE = 16
NC_SC = 2
NS_SC = 16
L = 16

IW = 80
BLK = 160
NCH = BLK // IW
EPT = E // NS_SC
NBLK = EPT // BLK
N_PAD = 10240
ROWS_PT = N_PAD // NS_SC

BE = 2000
BN_BLK = 1000


def _edge_lin_body(attr_ref, w_ref, b_ref, e0_ref, e1_ref):
    r = jnp.dot(attr_ref[...], w_ref[...], preferred_element_type=jnp.float32)
    r = r + b_ref[...]
    e0_ref[...] = r[:, :DH]
    e1_ref[...] = r[:, DH:]


def _edge_linear(edge_attr, w_edge, b_edge):
    return pl.pallas_call(
        _edge_lin_body,
        grid=(E // BE,),
        in_specs=[
            pl.BlockSpec((BE, DE), lambda i: (i, 0)),
            pl.BlockSpec((DE, D), lambda i: (0, 0)),
            pl.BlockSpec((1, D), lambda i: (0, 0)),
        ],
        out_specs=[
            pl.BlockSpec((BE, DH), lambda i: (i, 0)),
            pl.BlockSpec((BE, DH), lambda i: (i, 0)),
        ],
        out_shape=[
            jax.ShapeDtypeStruct((E, DH), jnp.float32),
            jax.ShapeDtypeStruct((E, DH), jnp.float32),
        ],
    )(edge_attr, w_edge, b_edge)


@functools.cache
def _build_sc_aggregate():
    mesh = plsc.VectorSubcoreMesh(core_axis_name="c", subcore_axis_name="s",
                                  num_cores=NC_SC, num_subcores=NS_SC)
    return pl.kernel(
        _sc_body,
        out_type=jax.ShapeDtypeStruct((NC_SC, N_PAD, D), jnp.float32),
        mesh=mesh,
        scratch_types=[
            pltpu.VMEM_SHARED((N_PAD, D), jnp.float32),
            pltpu.VMEM((NCH, IW), jnp.int32),
            pltpu.VMEM((NCH, IW), jnp.int32),
            pltpu.VMEM((BLK, D), jnp.float32),
            pltpu.VMEM((BLK, DH), jnp.float32),
            pltpu.VMEM((L,), jnp.float32),
            pltpu.SemaphoreType.DMA,
        ],
    )


def _sc_body(x_hbm, e0_hbm, e1_hbm, src_hbm, dst_hbm, t_hbm,
             acc_hbm, acc_sh, si, di, xg, eb, tv, sem):
    c = lax.axis_index("c")
    s = lax.axis_index("s")
    pltpu.sync_copy(t_hbm, tv)

    def _zrow(i, _):
        for k in range(D // L):
            xg[i, pl.ds(k * L, L)] = jnp.zeros((L,), jnp.float32)
        return 0
    lax.fori_loop(0, BLK, _zrow, 0)

    r0 = s * ROWS_PT
    off = 0
    for sz in _chunks(ROWS_PT, BLK):
        pltpu.sync_copy(xg.at[pl.ds(0, sz), :], acc_sh.at[pl.ds(r0 + off, sz), :])
        off += sz
    plsc.subcore_barrier()

    tvv = tv[...]
    blk0 = s * NBLK
    col = c * DH

    def _block(bi, _):
        blk = blk0 + bi
        pltpu.sync_copy(src_hbm.at[blk], si)
        pltpu.sync_copy(dst_hbm.at[blk], di)

        cps = [
            pltpu.async_copy(x_hbm.at[si.at[ch]],
                             xg.at[pl.ds(ch * IW, IW), :], sem)
            for ch in range(NCH)
        ]
        for cp in cps:
            cp.wait()

        @pl.when(c == 0)
        def _():
            pltpu.sync_copy(e0_hbm.at[pl.ds(s * EPT + bi * BLK, BLK), :], eb)

        @pl.when(c == 1)
        def _():
            pltpu.sync_copy(e1_hbm.at[pl.ds(s * EPT + bi * BLK, BLK), :], eb)

        def _row(i, _):
            for k in range(DH // L):
                m = jnp.maximum(xg[i, pl.ds(col + k * L, L)]
                                + eb[i, pl.ds(k * L, L)], 0.0) + 1e-7
                ex = jnp.exp(m * tvv)
                xg[i, pl.ds(k * L, L)] = ex * m
                xg[i, pl.ds(DH + k * L, L)] = ex
            return 0
        lax.fori_loop(0, BLK, _row, 0)

        for ch in range(NCH):
            pltpu.sync_copy(xg.at[pl.ds(ch * IW, IW), :],
                            acc_sh.at[di.at[ch]], add=True)
        return 0

    lax.fori_loop(0, NBLK, _block, 0)
    plsc.subcore_barrier()

    off = 0
    for sz in _chunks(ROWS_PT, BLK):
        pltpu.sync_copy(acc_sh.at[pl.ds(r0 + off, sz), :], xg.at[pl.ds(0, sz), :])
        pltpu.sync_copy(xg.at[pl.ds(0, sz), :],
                        acc_hbm.at[c, pl.ds(r0 + off, sz), :])
        off += sz


def _chunks(total, step):
    sizes = [step] * (total // step)
    if total % step:
        sizes.append(total % step)
    return sizes


def _mlp_body(x_ref, nu_ref, de_ref, w1_ref, b1_ref, w2_ref, b2_ref,
              h2_ref, s_ref):
    i = pl.program_id(0)
    aggr = nu_ref[...] / (de_ref[...] + 1e-16)
    h = x_ref[...] + aggr
    h1 = jnp.maximum(
        jnp.dot(h, w1_ref[...], preferred_element_type=jnp.float32)
        + b1_ref[...], 0.0)
    h2 = jnp.dot(h1, w2_ref[...], preferred_element_type=jnp.float32) + b2_ref[...]
    h2_ref[...] = h2
    acc = jnp.concatenate(
        [jnp.sum(h2, axis=0, keepdims=True),
         jnp.sum(h2 * h2, axis=0, keepdims=True),
         jnp.zeros((6, D), jnp.float32)], axis=0)

    @pl.when(i == 0)
    def _():
        s_ref[...] = acc

    @pl.when(i > 0)
    def _():
        s_ref[...] = s_ref[...] + acc


def _mlp_stage(x, numer, denom, w1, b1, w2, b2):
    return pl.pallas_call(
        _mlp_body,
        grid=(N // BN_BLK,),
        in_specs=[
            pl.BlockSpec((BN_BLK, D), lambda i: (i, 0)),
            pl.BlockSpec((BN_BLK, D), lambda i: (i, 0)),
            pl.BlockSpec((BN_BLK, D), lambda i: (i, 0)),
            pl.BlockSpec((D, 2 * D), lambda i: (0, 0)),
            pl.BlockSpec((1, 2 * D), lambda i: (0, 0)),
            pl.BlockSpec((2 * D, D), lambda i: (0, 0)),
            pl.BlockSpec((1, D), lambda i: (0, 0)),
        ],
        out_specs=[
            pl.BlockSpec((BN_BLK, D), lambda i: (i, 0)),
            pl.BlockSpec((8, D), lambda i: (0, 0)),
        ],
        out_shape=[
            jax.ShapeDtypeStruct((N, D), jnp.float32),
            jax.ShapeDtypeStruct((8, D), jnp.float32),
        ],
    )(x, numer, denom, w1, b1, w2, b2)


def _bn_silu_lin_body(h_ref, s_ref, g_ref, b_ref, w_ref, o_ref, s2_ref):
    i = pl.program_id(0)
    st = s_ref[...]
    mu = st[0:1, :] * (1.0 / N)
    var = st[1:2, :] * (1.0 / N) - mu * mu
    xn = (h_ref[...] - mu) * lax.rsqrt(var + 1e-5) * g_ref[...] + b_ref[...]
    g = xn * jax.nn.sigmoid(xn)
    out = jnp.dot(g, w_ref[...], preferred_element_type=jnp.float32)
    o_ref[...] = out
    acc = jnp.concatenate(
        [jnp.sum(out, axis=0, keepdims=True),
         jnp.sum(out * out, axis=0, keepdims=True),
         jnp.zeros((6, D), jnp.float32)], axis=0)

    @pl.when(i == 0)
    def _():
        s2_ref[...] = acc

    @pl.when(i > 0)
    def _():
        s2_ref[...] = s2_ref[...] + acc


def _bn_silu_lin_stage(h2, sums, gamma, beta, w):
    return pl.pallas_call(
        _bn_silu_lin_body,
        grid=(N // BN_BLK,),
        in_specs=[
            pl.BlockSpec((BN_BLK, D), lambda i: (i, 0)),
            pl.BlockSpec((8, D), lambda i: (0, 0)),
            pl.BlockSpec((1, D), lambda i: (0, 0)),
            pl.BlockSpec((1, D), lambda i: (0, 0)),
            pl.BlockSpec((D, D), lambda i: (0, 0)),
        ],
        out_specs=[
            pl.BlockSpec((BN_BLK, D), lambda i: (i, 0)),
            pl.BlockSpec((8, D), lambda i: (0, 0)),
        ],
        out_shape=[
            jax.ShapeDtypeStruct((N, D), jnp.float32),
            jax.ShapeDtypeStruct((8, D), jnp.float32),
        ],
    )(h2, sums, gamma, beta, w)


def _bn_silu_body(h_ref, s_ref, g_ref, b_ref, o_ref):
    st = s_ref[...]
    mu = st[0:1, :] * (1.0 / N)
    var = st[1:2, :] * (1.0 / N) - mu * mu
    xn = (h_ref[...] - mu) * lax.rsqrt(var + 1e-5) * g_ref[...] + b_ref[...]
    o_ref[...] = xn * jax.nn.sigmoid(xn)


def _bn_silu_stage(g, sums, gamma, beta):
    return pl.pallas_call(
        _bn_silu_body,
        grid=(N // BN_BLK,),
        in_specs=[
            pl.BlockSpec((BN_BLK, D), lambda i: (i, 0)),
            pl.BlockSpec((8, D), lambda i: (0, 0)),
            pl.BlockSpec((1, D), lambda i: (0, 0)),
            pl.BlockSpec((1, D), lambda i: (0, 0)),
        ],
        out_specs=pl.BlockSpec((BN_BLK, D), lambda i: (i, 0)),
        out_shape=jax.ShapeDtypeStruct((N, D), jnp.float32),
    )(g, sums, gamma, beta)


def kernel(x, edge_index, edge_attr, W_edge, b_edge, t,
           mlp_W1, mlp_b1, mlp_W2, mlp_b2,
           bn1_gamma, bn1_beta, lin_W, bn2_gamma, bn2_beta):
    x = x.astype(jnp.float32)
    src3 = edge_index[0].astype(jnp.int32).reshape(E // BLK, NCH, IW)
    dst3 = edge_index[1].astype(jnp.int32).reshape(E // BLK, NCH, IW)
    tvec = jnp.full((L,), 1.0, jnp.float32) * t.astype(jnp.float32)

    e0, e1 = _edge_linear(edge_attr, W_edge, b_edge.reshape(1, D))
    acc = _build_sc_aggregate()(x, e0, e1, src3, dst3, tvec)
    numer = jnp.concatenate([acc[0, :N, :DH], acc[1, :N, :DH]], axis=1)
    denom = jnp.concatenate([acc[0, :N, DH:], acc[1, :N, DH:]], axis=1)

    h2, s1 = _mlp_stage(x, numer, denom, mlp_W1, mlp_b1.reshape(1, 2 * D),
                        mlp_W2, mlp_b2.reshape(1, D))
    g, s2 = _bn_silu_lin_stage(h2, s1, bn1_gamma.reshape(1, D),
                               bn1_beta.reshape(1, D), lin_W)
    return _bn_silu_stage(g, s2, bn2_gamma.reshape(1, D),
                          bn2_beta.reshape(1, D))
